# baseline (speedup 1.0000x reference)
import functools

import jax
import jax.numpy as jnp
from jax import lax
from jax.experimental import pallas as pl
from jax.experimental.pallas import tpu as pltpu
from jax.experimental.pallas import tpu_sc as plsc

BATCH = 16384
NUM_ROWS = 1000000
EMBED_DIM = 32
LANES = 16
NUM_CORES = 2
NUM_SUBCORES = 16
NUM_WORKERS = NUM_CORES * NUM_SUBCORES
BPW = BATCH // NUM_WORKERS
LANE_TILE = 128

S_STRIPE = 32768
CHUNK_U = 1024
NCHUNKS = S_STRIPE // CHUNK_U
TPC = CHUNK_U // LANE_TILE
NSG = 8
GPS = 7
HITCAP = NSG * GPS * LANES
STAGE_ROWS = BATCH + 1
TRASH = BATCH
LAST_TILE = (NUM_ROWS // LANE_TILE) * LANE_TILE
SLAB_ROWS = 128
FLUSH_AT = SLAB_ROWS - LANES


def _build_sweep():
    mesh = plsc.VectorSubcoreMesh(core_axis_name="c", subcore_axis_name="s")

    @functools.partial(
        pl.kernel,
        mesh=mesh,
        out_type=[jax.ShapeDtypeStruct((STAGE_ROWS, LANE_TILE), jnp.float32),
                  jax.ShapeDtypeStruct((STAGE_ROWS, LANE_TILE), jnp.float32)],
        compiler_params=pltpu.CompilerParams(
            needs_layout_passes=False, use_tc_tiling_on_sc=True),
        scratch_types=[
            pltpu.VMEM((BATCH,), jnp.int32),
            pltpu.VMEM((BATCH,), jnp.int32),
            pltpu.VMEM((TPC * EMBED_DIM, LANE_TILE), jnp.float32),
            pltpu.VMEM((SLAB_ROWS, LANE_TILE), jnp.float32),
            pltpu.VMEM((1, SLAB_ROWS), jnp.int32),
            pltpu.VMEM((HITCAP,), jnp.int32),
            pltpu.VMEM((HITCAP,), jnp.int32),
            pltpu.VMEM((HITCAP,), jnp.int32),
            pltpu.VMEM((HITCAP,), jnp.int32),
            pltpu.SemaphoreType.DMA,
            pltpu.SemaphoreType.DMA,
        ],
    )
    def sweep_kernel(uidx_hbm, iidx_hbm, utab_hbm, ttab_hbm,
                     ustage_hbm, tstage_hbm,
                     uidx_v, iidx_v, chunk_v, row_v, b_v,
                     hu_u, hu_b, hi_u, hi_b, sem_f, sem_s):
        wid = lax.axis_index("s") * NUM_CORES + lax.axis_index("c")
        lo = wid * S_STRIPE

        pltpu.sync_copy(uidx_hbm, uidx_v)
        pltpu.sync_copy(iidx_hbm, iidx_v)

        iota16 = lax.iota(jnp.int32, LANES)
        trash16 = jnp.full((LANES,), TRASH, jnp.int32)

        def scan(idx_v, h_u, h_b):
            def body(c, off):
                vec = idx_v[pl.ds(c * LANES, LANES)]
                m = (vec >= lo) & (vec < lo + S_STRIPE)
                n = jnp.sum(m.astype(jnp.int32))
                plsc.store_compressed(h_u.at[pl.ds(off, LANES)], vec, mask=m)
                plsc.store_compressed(h_b.at[pl.ds(off, LANES)],
                                      c * LANES + iota16, mask=m)
                return off + n
            return lax.fori_loop(0, BATCH // LANES, body, 0)

        nu = scan(uidx_v, hu_u, hu_b)
        ni = scan(iidx_v, hi_u, hi_b)

        def prefill_trash():
            for s in range(SLAB_ROWS // LANES):
                b_v[0, pl.ds(s * LANES, LANES)] = trash16

        def sweep_one(tab_hbm, h_u, h_b, nh, stage_hbm):
            prefill_trash()

            def flush(stage_hbm):
                pltpu.async_copy(row_v, stage_hbm.at[b_v.at[0]], sem_s).wait()
                prefill_trash()

            def chunk_body(cc, cnt):
                clo = lo + cc * CHUNK_U
                fets = []
                for tc in range(TPC):
                    st = jnp.minimum(clo + tc * LANE_TILE, LAST_TILE)
                    st = pl.multiple_of(st, LANE_TILE)
                    fets.append(pltpu.async_copy(
                        tab_hbm.at[:, pl.ds(st, LANE_TILE)],
                        chunk_v.at[pl.ds(tc * EMBED_DIM, EMBED_DIM)], sem_f))
                for f in fets:
                    f.wait()

                def sg_body(sg, cnt):
                    for i in range(GPS):
                        gbase = sg * (GPS * LANES) + i * LANES
                        hu = h_u[pl.ds(gbase, LANES)]
                        hb = h_b[pl.ds(gbase, LANES)]
                        m = ((gbase + iota16 < nh)
                             & (hu >= clo) & (hu < clo + CHUNK_U))
                        n = jnp.sum(m.astype(jnp.int32))

                        @pl.when(n > 0)
                        def _():
                            local = hu - clo
                            sub = local // LANE_TILE
                            lane = local - sub * LANE_TILE
                            pos = cnt + plsc.cumsum(m.astype(jnp.int32)) - 1
                            for d in range(EMBED_DIM):
                                rows = sub * EMBED_DIM + d
                                vals = plsc.load_gather(
                                    chunk_v, [rows, lane], mask=m)
                                plsc.store_scatter(
                                    row_v,
                                    [pos, jnp.full((LANES,), d, jnp.int32)],
                                    vals, mask=m)
                            plsc.store_scatter(
                                b_v, [jnp.zeros((LANES,), jnp.int32), pos],
                                hb, mask=m)

                        cnt = cnt + n

                        @pl.when(cnt >= FLUSH_AT)
                        def _():
                            flush(stage_hbm)

                        cnt = jnp.where(cnt >= FLUSH_AT, 0, cnt)
                    return cnt

                return lax.fori_loop(0, NSG, sg_body, cnt)

            lax.fori_loop(0, NCHUNKS, chunk_body, 0)
            flush(stage_hbm)

        sweep_one(utab_hbm, hu_u, hu_b, nu, ustage_hbm)
        sweep_one(ttab_hbm, hi_u, hi_b, ni, tstage_hbm)

    return sweep_kernel


def _build_dot():
    mesh = plsc.VectorSubcoreMesh(core_axis_name="c", subcore_axis_name="s")
    BLK = 128

    @functools.partial(
        pl.kernel,
        mesh=mesh,
        out_type=jax.ShapeDtypeStruct((BATCH,), jnp.float32),
        compiler_params=pltpu.CompilerParams(
            needs_layout_passes=False, use_tc_tiling_on_sc=False),
        scratch_types=[
            pltpu.VMEM((BLK, LANE_TILE), jnp.float32),
            pltpu.VMEM((BLK, LANE_TILE), jnp.float32),
            pltpu.VMEM((BPW,), jnp.float32),
            pltpu.SemaphoreType.DMA,
        ],
    )
    def dot_kernel(ustage_hbm, tstage_hbm, out_hbm, ub_v, tb_v, out_v, sem):
        wid = lax.axis_index("s") * NUM_CORES + lax.axis_index("c")
        base = wid * BPW
        iota16 = lax.iota(jnp.int32, LANES)

        def blk_body(blk, _):
            off = base + blk * BLK
            cu = pltpu.async_copy(ustage_hbm.at[pl.ds(off, BLK)], ub_v, sem)
            ct = pltpu.async_copy(tstage_hbm.at[pl.ds(off, BLK)], tb_v, sem)
            cu.wait()
            ct.wait()
            for g in range(BLK // LANES):
                rows = g * LANES + iota16
                acc = jnp.zeros((LANES,), jnp.float32)
                for d in range(EMBED_DIM):
                    col = jnp.full((LANES,), d, jnp.int32)
                    u = plsc.load_gather(ub_v, [rows, col])
                    v = plsc.load_gather(tb_v, [rows, col])
                    acc = acc + u * v
                out_v[pl.ds(blk * BLK + g * LANES, LANES)] = (
                    1.0 / (1.0 + jnp.exp(-acc)))
            return 0

        lax.fori_loop(0, BPW // BLK, blk_body, 0)
        pltpu.sync_copy(out_v, out_hbm.at[pl.ds(base, BPW)])

    return dot_kernel


@functools.cache
def _get_sweep():
    return _build_sweep()


@functools.cache
def _get_dot():
    return _build_dot()


def kernel(user_indices, item_indices, user_table, src_item_table, tgt_item_table):
    del src_item_table
    ustage, tstage = _get_sweep()(user_indices.astype(jnp.int32),
                                  item_indices.astype(jnp.int32),
                                  user_table.T, tgt_item_table.T)
    return _get_dot()(ustage, tstage)


# speedup vs baseline: 2.5621x; 2.5621x over previous
import functools

import jax
import jax.numpy as jnp
from jax import lax
from jax.experimental import pallas as pl
from jax.experimental.pallas import tpu as pltpu
from jax.experimental.pallas import tpu_sc as plsc

BATCH = 16384
EMBED_DIM = 32
LANES = 16
NUM_CORES = 2
NUM_SUBCORES = 16
NUM_WORKERS = NUM_CORES * NUM_SUBCORES
BPW = BATCH // NUM_WORKERS
LANE_TILE = 128
CHUNKS = BPW // LANES


def _build():
    mesh = plsc.VectorSubcoreMesh(core_axis_name="c", subcore_axis_name="s")

    @functools.partial(
        pl.kernel,
        mesh=mesh,
        out_type=jax.ShapeDtypeStruct((BATCH,), jnp.float32),
        compiler_params=pltpu.CompilerParams(
            needs_layout_passes=False, use_tc_tiling_on_sc=True),
        scratch_types=[
            pltpu.VMEM((BPW,), jnp.int32),
            pltpu.VMEM((BPW,), jnp.int32),
            pltpu.VMEM((24 * EMBED_DIM, LANE_TILE), jnp.float32),
            pltpu.VMEM((EMBED_DIM * LANES,), jnp.float32),
            pltpu.VMEM((EMBED_DIM * LANES,), jnp.float32),
            pltpu.VMEM((BPW,), jnp.float32),
            pltpu.SemaphoreType.DMA,
            pltpu.SemaphoreType.DMA,
        ],
    )
    def cmf_kernel(uidx_hbm, iidx_hbm, utab_hbm, ttab_hbm, out_hbm,
                   uidx_v, iidx_v, tiles_v, ucomp_v, tcomp_v, out_v,
                   sem_u, sem_i):
        wid = lax.axis_index("s") * NUM_CORES + lax.axis_index("c")
        base = wid * BPW

        pltpu.sync_copy(uidx_hbm.at[pl.ds(base, BPW)], uidx_v)
        pltpu.sync_copy(iidx_hbm.at[pl.ds(base, BPW)], iidx_v)

        iota16 = lax.iota(jnp.int32, LANES)
        slot_rows = iota16 * EMBED_DIM
        lane_mod = jnp.full((LANES,), LANE_TILE, jnp.int32)

        def fetch_and_extract(tab_hbm, idx_ref, comp_ref, c, sem):
            vec = idx_ref[pl.ds(c * LANES, LANES)]
            copies = []
            for k in range(LANES):
                tcol = vec[k] // LANE_TILE
                start = pl.multiple_of(tcol * LANE_TILE, LANE_TILE)
                copies.append(pltpu.async_copy(
                    tab_hbm.at[:, pl.ds(start, LANE_TILE)],
                    tiles_v.at[pl.ds(k * EMBED_DIM, EMBED_DIM)], sem))
            for cp in copies:
                cp.wait()
            lane = lax.rem(vec, lane_mod)
            for d in range(EMBED_DIM):
                vals = plsc.load_gather(tiles_v, [slot_rows + d, lane])
                comp_ref[pl.ds(d * LANES, LANES)] = vals

        def chunk_body(c, _):
            fetch_and_extract(utab_hbm, uidx_v, ucomp_v, c, sem_u)
            fetch_and_extract(ttab_hbm, iidx_v, tcomp_v, c, sem_i)
            acc = jnp.zeros((LANES,), jnp.float32)
            for d in range(EMBED_DIM):
                u = ucomp_v[pl.ds(d * LANES, LANES)]
                v = tcomp_v[pl.ds(d * LANES, LANES)]
                acc = acc + u * v
            out_v[pl.ds(c * LANES, LANES)] = 1.0 / (1.0 + jnp.exp(-acc))
            return 0

        lax.fori_loop(0, CHUNKS, chunk_body, 0)

        pltpu.sync_copy(out_v, out_hbm.at[pl.ds(base, BPW)])

    return cmf_kernel


@functools.cache
def _get_cmf():
    return _build()


def kernel(user_indices, item_indices, user_table, src_item_table, tgt_item_table):
    del src_item_table
    return _get_cmf()(user_indices.astype(jnp.int32),
                      item_indices.astype(jnp.int32),
                      user_table.T, tgt_item_table.T)
